# baseline (speedup 1.0000x reference)
import functools

import jax
import jax.numpy as jnp
from jax import lax
from jax.experimental import pallas as pl
from jax.experimental.pallas import tpu as pltpu
from jax.experimental.pallas import tpu_sc as plsc

B = 4
N_IN = 112 * 112 * 192
N_OUT = 224 * 224 * 192
S = 16
CS = 21 * 65536
NCH = 7
NR = 21
BLK = NR * 128
NBLK = N_IN // (S * BLK)
PT = CS // S
WIN = 2048
NWIN = PT // WIN
SLOTS = 4
NG = NBLK // SLOTS
NTASK = B * NCH // 2


def _body(mask_hbm, upd_hbm, out_hbm,
          i0, i1, i2, i3, v0, v1, v2, v3, ramp, zsrc, w0, w1, acc,
          l0, l1, l2, l3, s0, s1, s2, s3, zsem):
    idx = [i0, i1, i2, i3]
    val = [v0, v1, v2, v3]
    lsem = [l0, l1, l2, l3]
    ssem = [s0, s1, s2, s3]
    wbufs = [w0, w1]
    cid = lax.axis_index("c")
    sid = lax.axis_index("s")

    for q in range(16):
        for k in range(8):
            ramp[q, pl.ds(k * 16, 16)] = (q * 128 + k * 16
                                          + lax.iota(jnp.int32, 16))
    for k in range(8):
        zsrc[pl.ds(k * 16, 16)] = jnp.zeros((16,), jnp.float32)

    def window(w):
        return acc.at[pl.ds(sid * PT + w * WIN, WIN)]

    def load(b, n, s, started):
        d0 = pltpu.make_async_copy(mask_hbm.at[b, sid, n], idx[s], lsem[s])
        d1 = pltpu.make_async_copy(upd_hbm.at[b, sid, n], val[s], lsem[s])
        if started:
            d0.wait()
            d1.wait()
        else:
            d0.start()
            d1.start()

    def scat_drain(s, c):
        def one(r, cc):
            pltpu.make_async_copy(
                val[s].at[r],
                acc.at[plsc.Indices(idx[s].at[r], ignored_value=-1)],
                ssem[s]).wait()
            return cc
        return lax.fori_loop(0, NR, one, c)

    def task(t, c):
        tid = 2 * t + cid
        b = tid // NCH
        ch = tid - NCH * b
        base = ch * CS

        def zgroup(g, cc):
            for p in range(2):
                w = 2 * g + p
                for q in range(16):
                    pltpu.make_async_copy(
                        zsrc, window(w).at[plsc.Indices(ramp.at[q])],
                        ssem[p]).start()

                @pl.when(w >= 2)
                def _():
                    def zdrain(q, qc):
                        pltpu.make_async_copy(
                            zsrc,
                            window(w - 2).at[plsc.Indices(ramp.at[q])],
                            ssem[p]).wait()
                        return qc
                    lax.fori_loop(0, 16, zdrain, 0)
            return cc

        lax.fori_loop(0, NWIN // 2, zgroup, 0)
        for w in (NWIN - 2, NWIN - 1):
            def zdrain2(q, qc):
                pltpu.make_async_copy(
                    zsrc, window(w).at[plsc.Indices(ramp.at[q])],
                    ssem[w % 2]).wait()
                return qc
            lax.fori_loop(0, 16, zdrain2, 0)
        plsc.subcore_barrier()

        load(b, 0, 0, False)
        load(b, 1, 1, False)

        def group(g, cc):
            for s in range(SLOTS):
                n = SLOTS * g + s
                load(b, n, s, True)

                def row(r, rc):
                    for k in range(8):
                        iv = idx[s][r, pl.ds(k * 16, 16)]
                        u = iv - base
                        m = plsc.bitcast(u, jnp.uint32) < jnp.uint32(CS)
                        idx[s][r, pl.ds(k * 16, 16)] = jnp.where(
                            m, u, jnp.int32(-1))
                    pltpu.async_copy(
                        val[s].at[r],
                        acc.at[plsc.Indices(idx[s].at[r], ignored_value=-1)],
                        ssem[s], add=True)
                    return rc

                lax.fori_loop(0, NR, row, 0)

                s2 = (s + 2) % SLOTS

                @pl.when(n >= 2)
                def _():
                    scat_drain(s2, 0)

                @pl.when(n <= NBLK - 3)
                def _():
                    load(b, n + 2, s2, False)
            return cc

        lax.fori_loop(0, NG, group, 0)
        scat_drain(2, 0)
        scat_drain(3, 0)
        plsc.subcore_barrier()

        def store(w, p):
            return pltpu.make_async_copy(
                wbufs[p],
                out_hbm.at[b, ch, pl.ds(sid * PT + w * WIN, WIN)],
                ssem[p])

        def wgroup(g, cc):
            for p in range(2):
                w = 2 * g + p

                @pl.when(w >= 2)
                def _():
                    store(w - 2, p).wait()

                for q in range(16):
                    pltpu.make_async_copy(
                        window(w).at[plsc.Indices(ramp.at[q])],
                        wbufs[p].at[pl.ds(q * 128, 128)],
                        lsem[p]).start()

                def gdrain(q, qc):
                    pltpu.make_async_copy(
                        window(w).at[plsc.Indices(ramp.at[q])],
                        wbufs[p].at[pl.ds(q * 128, 128)],
                        lsem[p]).wait()
                    return qc

                lax.fori_loop(0, 16, gdrain, 0)
                store(w, p).start()
            return cc

        lax.fori_loop(0, NWIN // 2, wgroup, 0)
        store(NWIN - 2, 0).wait()
        store(NWIN - 1, 1).wait()
        return c

    lax.fori_loop(0, NTASK, task, 0)


@jax.jit
def _unpool(mask_r, upd_r):
    f = functools.partial(
        pl.kernel,
        mesh=plsc.VectorSubcoreMesh(core_axis_name="c", subcore_axis_name="s"),
        out_type=jax.ShapeDtypeStruct((B, NCH, CS), jnp.float32),
        scratch_types=[
            pltpu.VMEM((NR, 128), jnp.int32),
            pltpu.VMEM((NR, 128), jnp.int32),
            pltpu.VMEM((NR, 128), jnp.int32),
            pltpu.VMEM((NR, 128), jnp.int32),
            pltpu.VMEM((NR, 128), jnp.float32),
            pltpu.VMEM((NR, 128), jnp.float32),
            pltpu.VMEM((NR, 128), jnp.float32),
            pltpu.VMEM((NR, 128), jnp.float32),
            pltpu.VMEM((16, 128), jnp.int32),
            pltpu.VMEM((128,), jnp.float32),
            pltpu.VMEM((WIN,), jnp.float32),
            pltpu.VMEM((WIN,), jnp.float32),
            pltpu.VMEM_SHARED((CS,), jnp.float32),
            pltpu.SemaphoreType.DMA,
            pltpu.SemaphoreType.DMA,
            pltpu.SemaphoreType.DMA,
            pltpu.SemaphoreType.DMA,
            pltpu.SemaphoreType.DMA,
            pltpu.SemaphoreType.DMA,
            pltpu.SemaphoreType.DMA,
            pltpu.SemaphoreType.DMA,
            pltpu.SemaphoreType.DMA,
        ],
    )(_body)
    return f(mask_r, upd_r)


def kernel(updates, mask):
    mask_r = mask.astype(jnp.int32).reshape(B, S, NBLK, NR, 128)
    upd_r = updates.reshape(B, S, NBLK, NR, 128)
    out = _unpool(mask_r, upd_r)
    return out.reshape(B, 224, 224, 192)


# speedup vs baseline: 1.1485x; 1.1485x over previous
import functools

import jax
import jax.numpy as jnp
from jax import lax
from jax.experimental import pallas as pl
from jax.experimental.pallas import tpu as pltpu
from jax.experimental.pallas import tpu_sc as plsc

B = 4
N_IN = 112 * 112 * 192
N_OUT = 224 * 224 * 192
S = 16
CS = 21 * 65536
NCH = 7
NR = 21
BLK = NR * 128
NBLK = N_IN // (S * BLK)
PT = CS // S
SLOTS = 4
NG = NBLK // SLOTS
NTASK = B * NCH // 2


def _body(zero_hbm, mask_hbm, upd_hbm, out_hbm,
          i0, i1, i2, i3, v0, v1, v2, v3, acc,
          l0, l1, l2, l3, s0, s1, s2, s3, zsem):
    idx = [i0, i1, i2, i3]
    val = [v0, v1, v2, v3]
    lsem = [l0, l1, l2, l3]
    ssem = [s0, s1, s2, s3]
    cid = lax.axis_index("c")
    sid = lax.axis_index("s")
    share = pl.ds(sid * PT, PT)

    def load(b, n, s, started):
        d0 = pltpu.make_async_copy(mask_hbm.at[b, sid, n], idx[s], lsem[s])
        d1 = pltpu.make_async_copy(upd_hbm.at[b, sid, n], val[s], lsem[s])
        if started:
            d0.wait()
            d1.wait()
        else:
            d0.start()
            d1.start()

    def scat_drain(s, c):
        def one(r, cc):
            pltpu.make_async_copy(
                val[s].at[r],
                acc.at[plsc.Indices(idx[s].at[r], ignored_value=-1)],
                ssem[s]).wait()
            return cc
        return lax.fori_loop(0, NR, one, c)

    def wback(tid):
        b = tid // NCH
        ch = tid - NCH * b
        return pltpu.make_async_copy(acc.at[share],
                                     out_hbm.at[b, ch, share], zsem)

    def task(t, c):
        tid = 2 * t + cid
        b = tid // NCH
        ch = tid - NCH * b
        base = ch * CS

        load(b, 0, 0, False)
        load(b, 1, 1, False)

        @pl.when(t > 0)
        def _():
            wback(tid - 2).wait()

        zd = pltpu.make_async_copy(zero_hbm.at[share], acc.at[share], zsem)
        zd.start()
        zd.wait()
        plsc.subcore_barrier()

        def group(g, cc):
            for s in range(SLOTS):
                n = SLOTS * g + s
                load(b, n, s, True)

                def row(r, rc):
                    for k in range(8):
                        iv = idx[s][r, pl.ds(k * 16, 16)]
                        u = iv - base
                        m = plsc.bitcast(u, jnp.uint32) < jnp.uint32(CS)
                        idx[s][r, pl.ds(k * 16, 16)] = jnp.where(
                            m, u, jnp.int32(-1))
                    pltpu.async_copy(
                        val[s].at[r],
                        acc.at[plsc.Indices(idx[s].at[r], ignored_value=-1)],
                        ssem[s], add=True)
                    return rc

                lax.fori_loop(0, NR, row, 0)

                s2 = (s + 2) % SLOTS

                @pl.when(n >= 2)
                def _():
                    scat_drain(s2, 0)

                @pl.when(n <= NBLK - 3)
                def _():
                    load(b, n + 2, s2, False)
            return cc

        lax.fori_loop(0, NG, group, 0)
        scat_drain(2, 0)
        scat_drain(3, 0)
        plsc.subcore_barrier()

        wback(tid).start()
        return c

    lax.fori_loop(0, NTASK, task, 0)
    wback(2 * (NTASK - 1) + cid).wait()


@jax.jit
def _unpool(zero_hbm, mask_r, upd_r):
    f = functools.partial(
        pl.kernel,
        mesh=plsc.VectorSubcoreMesh(core_axis_name="c", subcore_axis_name="s"),
        out_type=jax.ShapeDtypeStruct((B, NCH, CS), jnp.float32),
        scratch_types=[
            pltpu.VMEM((NR, 128), jnp.int32),
            pltpu.VMEM((NR, 128), jnp.int32),
            pltpu.VMEM((NR, 128), jnp.int32),
            pltpu.VMEM((NR, 128), jnp.int32),
            pltpu.VMEM((NR, 128), jnp.float32),
            pltpu.VMEM((NR, 128), jnp.float32),
            pltpu.VMEM((NR, 128), jnp.float32),
            pltpu.VMEM((NR, 128), jnp.float32),
            pltpu.VMEM_SHARED((CS,), jnp.float32),
            pltpu.SemaphoreType.DMA,
            pltpu.SemaphoreType.DMA,
            pltpu.SemaphoreType.DMA,
            pltpu.SemaphoreType.DMA,
            pltpu.SemaphoreType.DMA,
            pltpu.SemaphoreType.DMA,
            pltpu.SemaphoreType.DMA,
            pltpu.SemaphoreType.DMA,
            pltpu.SemaphoreType.DMA,
        ],
    )(_body)
    return f(zero_hbm, mask_r, upd_r)


def kernel(updates, mask):
    mask_r = mask.astype(jnp.int32).reshape(B, S, NBLK, NR, 128)
    upd_r = updates.reshape(B, S, NBLK, NR, 128)
    zero_hbm = jnp.zeros((CS,), jnp.float32)
    out = _unpool(zero_hbm, mask_r, upd_r)
    return out.reshape(B, 224, 224, 192)
